# Initial kernel scaffold; baseline (speedup 1.0000x reference)
import jax
import jax.numpy as jnp
from jax.experimental import pallas as pl


def kernel(z, codebook):
    raise NotImplementedError("write your pallas kernel here")


# speedup vs baseline: 1.3665x; 1.3665x over previous
import functools

import jax
import jax.numpy as jnp
from jax import lax
from jax.experimental import pallas as pl
from jax.experimental.pallas import tpu as pltpu

N_ROWS = 65536
N_CODES = 1024
DIM = 64
BLOCK = 1024
GRID = N_ROWS // BLOCK


def _tc_body(z_ref, cb_ref, idx_ref, loss_ref):
    zb = z_ref[...]
    cb = cb_ref[...]
    zsq = jnp.sum(zb * zb, axis=1)
    csq = jnp.sum(cb * cb, axis=1)
    prod = lax.dot_general(zb, cb, (((1,), (1,)), ((), ())),
                           preferred_element_type=jnp.float32)
    dist = (zsq[:, None] + csq[None, :]) - 2.0 * prod
    dmin = jnp.min(dist, axis=1)
    col_ids = lax.broadcasted_iota(jnp.int32, dist.shape, 1)
    idx = jnp.min(jnp.where(dist == dmin[:, None], col_ids, N_CODES), axis=1)
    idx_ref[0, 0, :] = idx

    @pl.when(pl.program_id(0) == 0)
    def _():
        loss_ref[...] = jnp.zeros((1, 1), jnp.float32)

    loss_ref[...] += jnp.sum(dmin).reshape(1, 1)


def _tc_argmin(z, codebook):
    idx3, loss_sum = pl.pallas_call(
        _tc_body,
        grid=(GRID,),
        in_specs=[
            pl.BlockSpec((BLOCK, DIM), lambda i: (i, 0)),
            pl.BlockSpec((N_CODES, DIM), lambda i: (0, 0)),
        ],
        out_specs=[
            pl.BlockSpec((1, 1, BLOCK), lambda i: (i, 0, 0)),
            pl.BlockSpec((1, 1), lambda i: (0, 0)),
        ],
        out_shape=[
            jax.ShapeDtypeStruct((GRID, 1, BLOCK), jnp.int32),
            jax.ShapeDtypeStruct((1, 1), jnp.float32),
        ],
    )(z, codebook)
    return idx3.reshape(N_ROWS), loss_sum[0, 0]


def _sc_gather(codebook, indices):
    from jax.experimental.pallas import tpu_sc as plsc

    info = plsc.get_sparse_core_info()
    nc, ns = info.num_cores, info.num_subcores
    nw = nc * ns
    b_per_w = N_ROWS // nw
    chunk = 512
    mesh = plsc.VectorSubcoreMesh(core_axis_name="c", subcore_axis_name="s")

    @functools.partial(
        pl.kernel, mesh=mesh,
        out_type=jax.ShapeDtypeStruct((N_ROWS, DIM), jnp.float32),
        compiler_params=pltpu.CompilerParams(use_tc_tiling_on_sc=False),
        scratch_types=[
            pltpu.VMEM((chunk,), jnp.int32),
            pltpu.VMEM((chunk, DIM), jnp.float32),
            pltpu.SemaphoreType.DMA,
        ],
    )
    def gather_k(table_hbm, idx_hbm, out_hbm, idx_v, rows_v, sem):
        wid = lax.axis_index("s") * nc + lax.axis_index("c")
        base = wid * b_per_w
        for c in range(b_per_w // chunk):
            off = base + c * chunk
            pltpu.sync_copy(idx_hbm.at[pl.ds(off, chunk)], idx_v)
            pltpu.async_copy(table_hbm.at[idx_v], rows_v, sem).wait()
            pltpu.sync_copy(rows_v, out_hbm.at[pl.ds(off, chunk)])

    return gather_k(codebook, indices)


def kernel(z, codebook):
    indices, loss_sum = _tc_argmin(z, codebook)
    quantized = _sc_gather(codebook, indices)
    vq_loss = loss_sum * jnp.float32(1.25 / (N_ROWS * DIM))
    return (quantized, indices, vq_loss)


# speedup vs baseline: 2.0790x; 1.5214x over previous
import functools

import jax
import jax.numpy as jnp
from jax import lax
from jax.experimental import pallas as pl
from jax.experimental.pallas import tpu as pltpu

N_ROWS = 65536
N_CODES = 1024
DIM = 64
BLOCK = 2048
GRID = N_ROWS // BLOCK


CHUNK_C = 128


def _tc_body(z_ref, cb_ref, idx_ref, loss_ref):
    zbt = z_ref[...].T
    cb = cb_ref[...]
    zsq = jnp.sum(zbt * zbt, axis=0)
    csq = jnp.sum(cb * cb, axis=1)
    cb2 = cb + cb
    run_min = None
    run_c = None
    for c in range(N_CODES // CHUNK_C):
        cb2c = cb2[c * CHUNK_C:(c + 1) * CHUNK_C, :]
        csq_c = csq[c * CHUNK_C:(c + 1) * CHUNK_C]
        prod2_c = lax.dot_general(cb2c, zbt, (((1,), (0,)), ((), ())),
                                  preferred_element_type=jnp.float32)
        dist_c = (zsq[None, :] + csq_c[:, None]) - prod2_c
        if c == 0:
            run_min = dist_c
            run_c = jnp.zeros(dist_c.shape, jnp.int32)
        else:
            better = dist_c < run_min
            run_min = jnp.where(better, dist_c, run_min)
            run_c = jnp.where(better, c, run_c)
    jmat = run_c * CHUNK_C + lax.broadcasted_iota(jnp.int32, run_c.shape, 0)
    dmin = jnp.min(run_min, axis=0)
    idx = jnp.min(jnp.where(run_min == dmin[None, :], jmat, N_CODES),
                  axis=0)
    idx_ref[0, 0, :] = idx

    @pl.when(pl.program_id(0) == 0)
    def _():
        loss_ref[...] = jnp.zeros((1, 1), jnp.float32)

    loss_ref[...] += jnp.sum(dmin).reshape(1, 1)


def _tc_argmin(z, codebook):
    idx3, loss_sum = pl.pallas_call(
        _tc_body,
        grid=(GRID,),
        in_specs=[
            pl.BlockSpec((BLOCK, DIM), lambda i: (i, 0)),
            pl.BlockSpec((N_CODES, DIM), lambda i: (0, 0)),
        ],
        out_specs=[
            pl.BlockSpec((1, 1, BLOCK), lambda i: (i, 0, 0)),
            pl.BlockSpec((1, 1), lambda i: (0, 0)),
        ],
        out_shape=[
            jax.ShapeDtypeStruct((GRID, 1, BLOCK), jnp.int32),
            jax.ShapeDtypeStruct((1, 1), jnp.float32),
        ],
    )(z, codebook)
    return idx3.reshape(N_ROWS), loss_sum[0, 0]


def _sc_gather(codebook, indices):
    from jax.experimental.pallas import tpu_sc as plsc

    info = plsc.get_sparse_core_info()
    nc, ns = info.num_cores, info.num_subcores
    nw = nc * ns
    b_per_w = N_ROWS // nw
    chunk = 512
    mesh = plsc.VectorSubcoreMesh(core_axis_name="c", subcore_axis_name="s")

    @functools.partial(
        pl.kernel, mesh=mesh,
        out_type=jax.ShapeDtypeStruct((N_ROWS, DIM), jnp.float32),
        compiler_params=pltpu.CompilerParams(use_tc_tiling_on_sc=False),
        scratch_types=[
            pltpu.VMEM((chunk,), jnp.int32),
            pltpu.VMEM((chunk, DIM), jnp.float32),
            pltpu.SemaphoreType.DMA,
        ],
    )
    def gather_k(table_hbm, idx_hbm, out_hbm, idx_v, rows_v, sem):
        wid = lax.axis_index("s") * nc + lax.axis_index("c")
        base = wid * b_per_w
        for c in range(b_per_w // chunk):
            off = base + c * chunk
            pltpu.sync_copy(idx_hbm.at[pl.ds(off, chunk)], idx_v)
            pltpu.async_copy(table_hbm.at[idx_v], rows_v, sem).wait()
            pltpu.sync_copy(rows_v, out_hbm.at[pl.ds(off, chunk)])

    return gather_k(codebook, indices)


def kernel(z, codebook):
    indices, loss_sum = _tc_argmin(z, codebook)
    quantized = _sc_gather(codebook, indices)
    vq_loss = loss_sum * jnp.float32(1.25 / (N_ROWS * DIM))
    return (quantized, indices, vq_loss)
